# baseline (speedup 1.0000x reference)
import functools

import jax
import jax.numpy as jnp
from jax import lax
from jax.experimental import pallas as pl
from jax.experimental.pallas import tpu as pltpu
from jax.experimental.pallas import tpu_sc as plsc

B = 128
D_MODEL = 768
VOCAB = 32000
E = 8
VBLK = 6400
HALF = VBLK // 2
V_TILES = VOCAB // VBLK


@functools.partial(
    pl.kernel,
    out_type=jax.ShapeDtypeStruct((B,), jnp.int32),
    mesh=plsc.VectorSubcoreMesh(core_axis_name="c", subcore_axis_name="s"),
    scratch_types=[
        pltpu.VMEM((B,), jnp.int32),
        pltpu.VMEM((B,), jnp.int32),
    ],
)
def _sc_route(p_hbm, e_hbm, p_v, e_v):
    @pl.when((lax.axis_index("c") == 0) & (lax.axis_index("s") == 0))
    def _():
        pltpu.sync_copy(p_hbm, p_v)
        for c in range(B // 16):
            sl = pl.ds(c * 16, 16)
            e_v[sl] = p_v[sl] % E
        pltpu.sync_copy(e_v, e_hbm)


def _moe_body(eidx_ref, x_ref, wa_ref, wb_ref, b_ref, o_ref):
    e = pl.program_id(1)
    mask = eidx_ref[:] == e
    acc_a = jax.lax.dot_general(
        x_ref[:], wa_ref[0],
        dimension_numbers=(((1,), (1,)), ((), ())),
        preferred_element_type=jnp.float32,
    )
    acc_b = jax.lax.dot_general(
        x_ref[:], wb_ref[0],
        dimension_numbers=(((1,), (1,)), ((), ())),
        preferred_element_type=jnp.float32,
    )
    acc = jnp.concatenate([acc_a, acc_b], axis=1) + b_ref[0]

    @pl.when(e == 0)
    def _():
        o_ref[:] = jnp.where(mask, acc, jnp.zeros_like(acc))

    @pl.when(e != 0)
    def _():
        o_ref[:] = jnp.where(mask, acc, o_ref[:])


def kernel(x, pointer_addresses, W, b):
    eidx = _sc_route(pointer_addresses.astype(jnp.int32))
    eidx2d = eidx.reshape(B, 1)
    out = pl.pallas_call(
        _moe_body,
        grid=(V_TILES, E),
        in_specs=[
            pl.BlockSpec((B, 1), lambda v, e: (0, 0)),
            pl.BlockSpec((B, D_MODEL), lambda v, e: (0, 0)),
            pl.BlockSpec((1, HALF, D_MODEL), lambda v, e: (e, 2 * v, 0)),
            pl.BlockSpec((1, HALF, D_MODEL), lambda v, e: (e, 2 * v + 1, 0)),
            pl.BlockSpec((1, 1, VBLK), lambda v, e: (e, 0, v)),
        ],
        out_specs=pl.BlockSpec((B, VBLK), lambda v, e: (0, v)),
        out_shape=jax.ShapeDtypeStruct((B, VOCAB), jnp.float32),
        compiler_params=pltpu.CompilerParams(
            dimension_semantics=("arbitrary", "arbitrary"),
        ),
    )(eidx2d, x, W, W, b.reshape(E, 1, VOCAB))
    return out


# speedup vs baseline: 1.0718x; 1.0718x over previous
import jax
import jax.numpy as jnp
from jax.experimental import pallas as pl
from jax.experimental.pallas import tpu as pltpu

B = 128
D_MODEL = 768
VOCAB = 32000
E = 8
VBLK = 6400
HALF = VBLK // 2
V_TILES = VOCAB // VBLK


def _moe_body(p_ref, x_ref, wa_ref, wb_ref, b_ref, o_ref):
    e = pl.program_id(1)
    mask = (p_ref[:] % E) == e
    xb = x_ref[:]
    acc_a = jax.lax.dot_general(
        xb, wa_ref[0],
        dimension_numbers=(((1,), (1,)), ((), ())),
        preferred_element_type=jnp.float32,
    )
    acc_b = jax.lax.dot_general(
        xb, wb_ref[0],
        dimension_numbers=(((1,), (1,)), ((), ())),
        preferred_element_type=jnp.float32,
    )
    acc = jnp.concatenate([acc_a, acc_b], axis=1) + b_ref[0]

    @pl.when(e == 0)
    def _():
        o_ref[:] = jnp.where(mask, acc, jnp.zeros_like(acc))

    @pl.when(e != 0)
    def _():
        o_ref[:] = jnp.where(mask, acc, o_ref[:])


def kernel(x, pointer_addresses, W, b):
    p2d = pointer_addresses.reshape(B, 1).astype(jnp.int32)
    out = pl.pallas_call(
        _moe_body,
        grid=(V_TILES, E),
        in_specs=[
            pl.BlockSpec((B, 1), lambda v, e: (0, 0)),
            pl.BlockSpec((B, D_MODEL), lambda v, e: (0, 0)),
            pl.BlockSpec((1, HALF, D_MODEL), lambda v, e: (e, 2 * v, 0)),
            pl.BlockSpec((1, HALF, D_MODEL), lambda v, e: (e, 2 * v + 1, 0)),
            pl.BlockSpec((1, 1, VBLK), lambda v, e: (e, 0, v)),
        ],
        out_specs=pl.BlockSpec((B, VBLK), lambda v, e: (0, v)),
        out_shape=jax.ShapeDtypeStruct((B, VOCAB), jnp.float32),
        compiler_params=pltpu.CompilerParams(
            dimension_semantics=("arbitrary", "arbitrary"),
        ),
    )(p2d, x, W, W, b.reshape(E, 1, VOCAB))
    return out
